# baseline (speedup 1.0000x reference)
import functools

import jax
import jax.numpy as jnp
from jax import lax
from jax.experimental import pallas as pl
from jax.experimental.pallas import tpu as pltpu
from jax.experimental.pallas import tpu_sc as plsc

B = 4096
L = 200
H = 64
NT = 32

_NC = 2
_NS = 16
_NW = _NC * _NS
_BPW = B // _NW


_NG = 13
_LP = _NG * 16


_DEPTH = 4


def _pool_body(idx_hbm, table_hbm, out_hbm, idx_v, ring_v, acc_v, *sems):
    wid = lax.axis_index("s") * _NC + lax.axis_index("c")
    base = wid * _BPW
    pltpu.sync_copy(idx_hbm.at[pl.ds(base, _BPW)], idx_v)

    def start(d, s):
        def body(g, carry):
            iv = idx_v[s, pl.ds(16 * g, 16)]
            pltpu.async_copy(
                table_hbm.at[iv], ring_v.at[d, pl.ds(16 * g, 16)], sems[d])
            return carry
        lax.fori_loop(0, _NG, body, 0)

    def wait(d):
        pltpu.make_async_copy(
            table_hbm.at[pl.ds(0, _LP)], ring_v.at[d], sems[d]).wait()

    zs = (jnp.zeros((16,), jnp.float32),) * 8

    def accum(d, s):
        def body(g, c):
            a = list(c)
            for j in range(16):
                half = (j % 2) * 4
                for k in range(4):
                    a[half + k] = a[half + k] + ring_v[d, 16 * g + j,
                                                       pl.ds(16 * k, 16)]
            return tuple(a)
        a = lax.fori_loop(0, _NG, body, zs)
        for k in range(4):
            acc_v[s, pl.ds(16 * k, 16)] = a[k] + a[4 + k]

    for d in range(_DEPTH):
        start(d, d)

    nblk = _BPW // _DEPTH

    def block(b, carry):
        for d in range(_DEPTH):
            s = _DEPTH * b + d
            wait(d)
            accum(d, s)

            @pl.when(b < nblk - 1)
            def _():
                start(d, s + _DEPTH)
        return carry

    lax.fori_loop(0, nblk, block, 0)
    pltpu.sync_copy(acc_v, out_hbm.at[pl.ds(base, _BPW)])


@functools.cache
def _pool():
    return pl.kernel(
        _pool_body,
        mesh=plsc.VectorSubcoreMesh(core_axis_name="c", subcore_axis_name="s"),
        compiler_params=pltpu.CompilerParams(use_tc_tiling_on_sc=False),
        out_type=jax.ShapeDtypeStruct((B, H), jnp.float32),
        scratch_types=[
            pltpu.VMEM((_BPW, _LP), jnp.int32),
            pltpu.VMEM((_DEPTH, _LP, H), jnp.float32),
            pltpu.VMEM((_BPW, H), jnp.float32),
        ] + [pltpu.SemaphoreType.DMA] * _DEPTH,
    )


def _softplus(x):
    return jnp.maximum(x, 0.0) + jnp.log1p(jnp.exp(-jnp.abs(x)))


def _dot_t(x, w):
    return lax.dot_general(x, w, (((1,), (1,)), ((), ())),
                           preferred_element_type=jnp.float32)


def _mlp_body(pooled_ref, rd_ref, W1a_ref, w1b_ref, b1_ref, W2_ref, b2_ref,
              Wmu_ref, bmu_ref, Wlv_ref, blv_ref, gmu_ref, betamu_ref,
              glv_ref, betalv_ref, loc_ref, scale_ref):
    rd = rd_ref[...]
    ave = pooled_ref[...] / rd
    lrd = jnp.log(rd)
    h = _dot_t(ave, W1a_ref[...]) + lrd * w1b_ref[...] + b1_ref[...]
    h = _softplus(h)
    h = _softplus(_dot_t(h, W2_ref[...]) + b2_ref[...])
    tl = _dot_t(h, Wmu_ref[...]) + bmu_ref[...]
    ts = _dot_t(h, Wlv_ref[...]) + blv_ref[...]
    eps = 1e-5
    ml = jnp.mean(tl, axis=0, keepdims=True)
    vl = jnp.mean((tl - ml) * (tl - ml), axis=0, keepdims=True)
    loc_ref[...] = (tl - ml) * lax.rsqrt(vl + eps) * gmu_ref[...] + betamu_ref[...]
    ms = jnp.mean(ts, axis=0, keepdims=True)
    vs = jnp.mean((ts - ms) * (ts - ms), axis=0, keepdims=True)
    scale_ref[...] = jnp.exp(
        0.5 * ((ts - ms) * lax.rsqrt(vs + eps) * glv_ref[...] + betalv_ref[...]))


_mlp = pl.pallas_call(
    _mlp_body,
    out_shape=(
        jax.ShapeDtypeStruct((B, NT), jnp.float32),
        jax.ShapeDtypeStruct((B, NT), jnp.float32),
    ),
)


def kernel(idx, read_depth, table, W1, b1, W2, b2, Wmu, bmu, Wlv, blv,
           gmu, betamu, glv, betalv):
    idx_pad = jnp.concatenate(
        [idx.astype(jnp.int32), jnp.zeros((B, _LP - L), jnp.int32)], axis=1)
    pooled = _pool()(idx_pad, table)
    return _mlp(pooled, read_depth,
                W1[:, :H], W1[:, H][None, :], b1[None, :],
                W2, b2[None, :],
                Wmu, bmu[None, :], Wlv, blv[None, :],
                gmu[None, :], betamu[None, :], glv[None, :], betalv[None, :])


# speedup vs baseline: 1.1129x; 1.1129x over previous
import functools

import jax
import jax.numpy as jnp
from jax import lax
from jax.experimental import pallas as pl
from jax.experimental.pallas import tpu as pltpu
from jax.experimental.pallas import tpu_sc as plsc

B = 4096
L = 200
H = 64
NT = 32

_NC = 2
_NS = 16
_NW = _NC * _NS
_BPW = B // _NW


_NG = 13
_LP = _NG * 16


_DEPTH = 4


def _pool_body(idx_hbm, table_hbm, out_hbm, idx_v, ring_v, acc_v, *sems):
    wid = lax.axis_index("s") * _NC + lax.axis_index("c")
    base = wid * _BPW
    pltpu.sync_copy(idx_hbm.at[pl.ds(base, _BPW)], idx_v)

    def start(d, s):
        def body(g, carry):
            iv = idx_v[s, pl.ds(16 * g, 16)]
            pltpu.async_copy(
                table_hbm.at[iv], ring_v.at[d, pl.ds(16 * g, 16)], sems[d])
            return carry
        lax.fori_loop(0, _NG, body, 0)

    def wait(d):
        pltpu.make_async_copy(
            table_hbm.at[pl.ds(0, _LP)], ring_v.at[d], sems[d]).wait()

    zs = (jnp.zeros((16,), jnp.float32),) * 8

    def accum(d, s):
        def widen_lo(x):
            u = plsc.bitcast(x, jnp.uint32)
            return plsc.bitcast(u << 16, jnp.float32)

        def widen_hi(x):
            u = plsc.bitcast(x, jnp.uint32)
            return plsc.bitcast(u & jnp.uint32(0xFFFF0000), jnp.float32)

        def body(g, c):
            a = list(c)
            for j in range(16):
                half = (j % 2) * 4
                for k in range(2):
                    x = ring_v[d, 16 * g + j, pl.ds(32 * k, 32)]
                    a[half + 2 * k] = a[half + 2 * k] + widen_lo(x)
                    a[half + 2 * k + 1] = a[half + 2 * k + 1] + widen_hi(x)
            return tuple(a)
        a = lax.fori_loop(0, _NG, body, zs)
        for k in range(4):
            acc_v[s, pl.ds(16 * k, 16)] = a[k] + a[4 + k]

    for d in range(_DEPTH):
        start(d, d)

    nblk = _BPW // _DEPTH

    def block(b, carry):
        for d in range(_DEPTH):
            s = _DEPTH * b + d
            wait(d)
            accum(d, s)

            @pl.when(b < nblk - 1)
            def _():
                start(d, s + _DEPTH)
        return carry

    lax.fori_loop(0, nblk, block, 0)
    pltpu.sync_copy(acc_v, out_hbm.at[pl.ds(base, _BPW)])


@functools.cache
def _pool():
    return pl.kernel(
        _pool_body,
        mesh=plsc.VectorSubcoreMesh(core_axis_name="c", subcore_axis_name="s"),
        compiler_params=pltpu.CompilerParams(
            use_tc_tiling_on_sc=False, needs_layout_passes=False),
        out_type=jax.ShapeDtypeStruct((B, H), jnp.float32),
        scratch_types=[
            pltpu.VMEM((_BPW, _LP), jnp.int32),
            pltpu.VMEM((_DEPTH, _LP, H), jnp.bfloat16),
            pltpu.VMEM((_BPW, H), jnp.float32),
        ] + [pltpu.SemaphoreType.DMA] * _DEPTH,
    )


def _softplus(x):
    return jnp.maximum(x, 0.0) + jnp.log1p(jnp.exp(-jnp.abs(x)))


def _dot_t(x, w):
    return lax.dot_general(x, w, (((1,), (1,)), ((), ())),
                           preferred_element_type=jnp.float32)


def _mlp_body(pooled_ref, rd_ref, W1a_ref, w1b_ref, b1_ref, W2_ref, b2_ref,
              Wmu_ref, bmu_ref, Wlv_ref, blv_ref, gmu_ref, betamu_ref,
              glv_ref, betalv_ref, loc_ref, scale_ref):
    rd = rd_ref[...]
    ave = pooled_ref[...] / rd
    lrd = jnp.log(rd)
    h = _dot_t(ave, W1a_ref[...]) + lrd * w1b_ref[...] + b1_ref[...]
    h = _softplus(h)
    h = _softplus(_dot_t(h, W2_ref[...]) + b2_ref[...])
    tl = _dot_t(h, Wmu_ref[...]) + bmu_ref[...]
    ts = _dot_t(h, Wlv_ref[...]) + blv_ref[...]
    eps = 1e-5
    ml = jnp.mean(tl, axis=0, keepdims=True)
    vl = jnp.mean((tl - ml) * (tl - ml), axis=0, keepdims=True)
    loc_ref[...] = (tl - ml) * lax.rsqrt(vl + eps) * gmu_ref[...] + betamu_ref[...]
    ms = jnp.mean(ts, axis=0, keepdims=True)
    vs = jnp.mean((ts - ms) * (ts - ms), axis=0, keepdims=True)
    scale_ref[...] = jnp.exp(
        0.5 * ((ts - ms) * lax.rsqrt(vs + eps) * glv_ref[...] + betalv_ref[...]))


_mlp = pl.pallas_call(
    _mlp_body,
    out_shape=(
        jax.ShapeDtypeStruct((B, NT), jnp.float32),
        jax.ShapeDtypeStruct((B, NT), jnp.float32),
    ),
)


def kernel(idx, read_depth, table, W1, b1, W2, b2, Wmu, bmu, Wlv, blv,
           gmu, betamu, glv, betalv):
    idx_pad = jnp.concatenate(
        [idx.astype(jnp.int32), jnp.zeros((B, _LP - L), jnp.int32)], axis=1)
    pooled_perm = _pool()(idx_pad, table.astype(jnp.bfloat16))
    pooled = jnp.transpose(
        pooled_perm.reshape(B, 2, 2, 16), (0, 1, 3, 2)).reshape(B, H)
    return _mlp(pooled, read_depth,
                W1[:, :H], W1[:, H][None, :], b1[None, :],
                W2, b2[None, :],
                Wmu, bmu[None, :], Wlv, blv[None, :],
                gmu[None, :], betamu[None, :], glv[None, :], betalv[None, :])
